# baseline (speedup 1.0000x reference)
import functools

import jax
import jax.numpy as jnp
from jax import lax
from jax.experimental import pallas as pl
from jax.experimental.pallas import tpu as pltpu
from jax.experimental.pallas import tpu_sc as plsc

_ROW_IDS = (5, 8, 7, 16, 256, 123)
_NUM_ROWS = len(_ROW_IDS)
_IDX_PADDED = jnp.array(_ROW_IDS + (0,) * (16 - _NUM_ROWS), dtype=jnp.int32)


def kernel(inputs):
    table_rows, d = inputs.shape

    mesh = plsc.VectorSubcoreMesh(core_axis_name="c", subcore_axis_name="s")

    @functools.partial(
        pl.kernel,
        mesh=mesh,
        out_type=jax.ShapeDtypeStruct((_NUM_ROWS, d), jnp.float32),
        scratch_types=[
            pltpu.VMEM((16,), jnp.int32),
            pltpu.VMEM((16, d), jnp.float32),
            pltpu.SemaphoreType.DMA,
        ],
    )
    def gather_rows(table_hbm, idx_hbm, out_hbm, idx_v, rows_v, sem):
        wid = lax.axis_index("s") * 2 + lax.axis_index("c")

        @pl.when(wid == 0)
        def _():
            pltpu.sync_copy(idx_hbm, idx_v)
            pltpu.async_copy(table_hbm.at[idx_v], rows_v, sem).wait()
            pltpu.sync_copy(rows_v.at[pl.ds(0, _NUM_ROWS)], out_hbm)

    return gather_rows(inputs, _IDX_PADDED)


# speedup vs baseline: 1.1410x; 1.1410x over previous
import functools

import jax
import jax.numpy as jnp
from jax.experimental import pallas as pl
from jax.experimental.pallas import tpu as pltpu
from jax.experimental.pallas import tpu_sc as plsc

_ROW_IDS = (5, 8, 7, 16, 256, 123)
_NUM_ROWS = len(_ROW_IDS)


def kernel(inputs):
    _, d = inputs.shape

    mesh = plsc.ScalarSubcoreMesh(axis_name="c", num_cores=2)

    @functools.partial(
        pl.kernel,
        mesh=mesh,
        out_type=jax.ShapeDtypeStruct((_NUM_ROWS, d), jnp.float32),
        scratch_types=[pltpu.SemaphoreType.DMA],
    )
    def gather_rows(table_hbm, out_hbm, sem):
        @pl.when(jax.lax.axis_index("c") == 0)
        def _():
            copies = [
                pltpu.make_async_copy(
                    table_hbm.at[pl.ds(row, 1)], out_hbm.at[pl.ds(i, 1)], sem
                )
                for i, row in enumerate(_ROW_IDS)
            ]
            for c in copies:
                c.start()
            for c in copies:
                c.wait()

    return gather_rows(inputs)


# speedup vs baseline: 1.2408x; 1.0875x over previous
import functools

import jax
import jax.numpy as jnp
from jax.experimental import pallas as pl
from jax.experimental.pallas import tpu as pltpu
from jax.experimental.pallas import tpu_sc as plsc

_ROW_IDS = (5, 8, 7, 16, 256, 123)
_NUM_ROWS = len(_ROW_IDS)


def kernel(inputs):
    _, d = inputs.shape

    mesh = plsc.ScalarSubcoreMesh(axis_name="c", num_cores=1)

    @functools.partial(
        pl.kernel,
        mesh=mesh,
        out_type=jax.ShapeDtypeStruct((_NUM_ROWS, d), jnp.float32),
        scratch_types=[pltpu.SemaphoreType.DMA],
    )
    def gather_rows(table_hbm, out_hbm, sem):
        copies = [
            pltpu.make_async_copy(
                table_hbm.at[pl.ds(row, 1)], out_hbm.at[pl.ds(i, 1)], sem
            )
            for i, row in enumerate(_ROW_IDS)
        ]
        for c in copies:
            c.start()
        for c in copies:
            c.wait()

    return gather_rows(inputs)
